# baseline (speedup 1.0000x reference)
import numpy as np
import jax
import jax.numpy as jnp
from jax import lax
from jax.experimental import pallas as pl
from jax.experimental.pallas import tpu as pltpu

_EPS = 1e-06


def _softplus(x):
    return jnp.maximum(x, 0.0) + jnp.log1p(jnp.exp(-jnp.abs(x)))


def _make_kernel(N, D, HIDDEN, RANK, TI, TJ, T):
    inv_count = 1.0 / (N + 1)

    def body(im_ref, jm_ref,
             rf_ref, attn_ref, W1a_ref, W1b_ref, w1c_ref, b1_ref,
             W2_ref, b2_ref, W3_ref, b3_ref, Wd_ref, bd_ref,
             sigma_ref, A_s, C_s, S_s, R8_s):
        t = pl.program_id(1)
        i = im_ref[t]
        j = jm_ref[t]

        @pl.when(t == 0)
        def _init():
            rf = rf_ref[0]
            A_s[...] = jnp.dot(rf, W1a_ref[...],
                               preferred_element_type=jnp.float32) + b1_ref[0]
            C_s[...] = jnp.dot(rf, W1b_ref[...],
                               preferred_element_type=jnp.float32)
            S_s[...] = jnp.zeros((N, HIDDEN), jnp.float32)
            R8_s[...] = jnp.zeros((N, 8, HIDDEN), jnp.float32)

        a = A_s[pl.ds(i * TI, TI), :]
        c = C_s[pl.ds(j * TJ, TJ), :]
        t1 = attn_ref[0][:, :, None] * w1c_ref[0][None, None, :]
        x1 = jnp.maximum(t1 + a[:, None, :] + c[None, :, :], 0.0)
        x2 = jnp.maximum(
            jnp.dot(x1.reshape(TI * TJ, HIDDEN), W2_ref[...],
                    preferred_element_type=jnp.float32)
            + b2_ref[0], 0.0).reshape(TI, TJ, HIDDEN)

        def _accumulate(get_tile):
            row_part = None
            cols = []
            for k in range(TJ // 8):
                tk = get_tile(k)
                row_part = tk if row_part is None else row_part + tk
                cols.append(jnp.sum(tk, axis=0))
            R8_s[pl.ds(i * TI, TI), :, :] += row_part
            S_s[pl.ds(j * TJ, TJ), :] += jnp.concatenate(cols, axis=0)

        @pl.when(i == j)
        def _diag_tile():
            rows = lax.broadcasted_iota(jnp.int32, (TI, 8), 0)
            cols = lax.broadcasted_iota(jnp.int32, (TI, 8), 1)
            _accumulate(lambda k: x2[:, 8 * k:8 * k + 8, :]
                        * ((cols + 8 * k) <= rows).astype(jnp.float32)[:, :, None])

        @pl.when(i != j)
        def _full_tile():
            _accumulate(lambda k: x2[:, 8 * k:8 * k + 8, :])

        @pl.when(t == T - 1)
        def _finish():
            s = S_s[...] + jnp.sum(R8_s[...], axis=1)
            u = (jnp.dot(s, W3_ref[...],
                         preferred_element_type=jnp.float32) * inv_count
                 + b3_ref[0])
            sig = lax.dot_general(u, u, (((1,), (1,)), ((), ())),
                                  preferred_element_type=jnp.float32)
            rf = rf_ref[0]
            d_raw = jnp.dot(rf, Wd_ref[...],
                            preferred_element_type=jnp.float32)[:, 0]
            d = _softplus(d_raw + bd_ref[0, 0]) + _EPS
            rr = lax.broadcasted_iota(jnp.int32, (N, N), 0)
            cc = lax.broadcasted_iota(jnp.int32, (N, N), 1)
            sigma_ref[0] = sig + jnp.where(rr == cc, d[:, None], 0.0)

    return body


def kernel(residue_features, attention, W1, b1, W2, b2, W3, b3, Wd, bd):
    B, N, D = residue_features.shape
    HIDDEN = W2.shape[0]
    RANK = W3.shape[1]
    TI = TJ = 256
    nI = N // TI
    nJ = N // TJ

    tril = [(i, j) for i in range(nI) for j in range(nJ) if j <= i]
    T = len(tril)
    imap = jnp.asarray(np.array([p[0] for p in tril], np.int32))
    jmap = jnp.asarray(np.array([p[1] for p in tril], np.int32))

    W1a = W1[:D]
    W1b = W1[D:2 * D]
    w1c = W1[2 * D].reshape(1, HIDDEN)
    b1r = b1.reshape(1, HIDDEN)
    b2r = b2.reshape(1, HIDDEN)
    b3r = b3.reshape(1, RANK)
    bdr = bd.reshape(1, 1)

    body = _make_kernel(N, D, HIDDEN, RANK, TI, TJ, T)

    full = lambda *shape: pl.BlockSpec(
        shape, lambda b, t, im, jm: (0,) * len(shape))

    grid_spec = pltpu.PrefetchScalarGridSpec(
        num_scalar_prefetch=2,
        grid=(B, T),
        in_specs=[
            pl.BlockSpec((1, N, D), lambda b, t, im, jm: (b, 0, 0)),
            pl.BlockSpec((1, TI, TJ),
                         lambda b, t, im, jm: (b, im[t], jm[t])),
            full(D, HIDDEN),
            full(D, HIDDEN),
            full(1, HIDDEN),
            full(1, HIDDEN),
            full(HIDDEN, HIDDEN),
            full(1, HIDDEN),
            full(HIDDEN, RANK),
            full(1, RANK),
            full(D, 1),
            full(1, 1),
        ],
        out_specs=pl.BlockSpec((1, N, N), lambda b, t, im, jm: (b, 0, 0)),
        scratch_shapes=[
            pltpu.VMEM((N, HIDDEN), jnp.float32),
            pltpu.VMEM((N, HIDDEN), jnp.float32),
            pltpu.VMEM((N, HIDDEN), jnp.float32),
            pltpu.VMEM((N, 8, HIDDEN), jnp.float32),
        ],
    )

    out = pl.pallas_call(
        body,
        grid_spec=grid_spec,
        out_shape=jax.ShapeDtypeStruct((B, N, N), jnp.float32),
        compiler_params=pltpu.CompilerParams(
            dimension_semantics=("arbitrary", "arbitrary"),
        ),
    )(imap, jmap,
      residue_features, attention, W1a, W1b, w1c, b1r,
      W2, b2r, W3, b3r, Wd, bdr)
    return out


# speedup vs baseline: 1.2019x; 1.2019x over previous
import numpy as np
import jax
import jax.numpy as jnp
from jax import lax
from jax.experimental import pallas as pl
from jax.experimental.pallas import tpu as pltpu

_EPS = 1e-06


def _softplus(x):
    return jnp.maximum(x, 0.0) + jnp.log1p(jnp.exp(-jnp.abs(x)))


def _make_kernel(N, D, HIDDEN, RANK, TI, TJ, T):
    inv_count = 1.0 / (N + 1)

    def body(im_ref, jm_ref,
             rf_ref, attn_ref, W1a_ref, W1b_ref, w1c_ref, b1_ref,
             W2_ref, b2_ref, W3_ref, b3_ref, Wd_ref, bd_ref,
             sigma_ref, A_s, C_s, S_s, R8_s):
        t = pl.program_id(1)
        i = im_ref[t]
        j = jm_ref[t]

        @pl.when(t == 0)
        def _init():
            rf = rf_ref[0]
            A_s[...] = jnp.dot(rf, W1a_ref[...],
                               preferred_element_type=jnp.float32) + b1_ref[0]
            C_s[...] = jnp.dot(rf, W1b_ref[...],
                               preferred_element_type=jnp.float32)
            S_s[...] = jnp.zeros((N, HIDDEN), jnp.float32)
            R8_s[...] = jnp.zeros((N, 8, HIDDEN), jnp.float32)

        a = A_s[pl.ds(i * TI, TI), :]
        c = C_s[pl.ds(j * TJ, TJ), :]
        t1 = attn_ref[0][:, :, None] * w1c_ref[0][None, None, :]
        x1 = jnp.maximum(t1 + a[:, None, :] + c[None, :, :], 0.0)
        x2 = jnp.maximum(
            jnp.dot(x1.reshape(TI * TJ, HIDDEN), W2_ref[...],
                    preferred_element_type=jnp.float32)
            + b2_ref[0], 0.0).reshape(TI, TJ, HIDDEN)

        def _accumulate(get_tile):
            for ic in range(TI // 8):
                racc = None
                for jc in range(TJ // 8):
                    tk = get_tile(ic, jc)
                    racc = tk if racc is None else racc + tk
                    S_s[pl.ds(j * TJ + jc * 8, 8), :] += jnp.sum(tk, axis=0)
                R8_s[pl.ds(i * TI + ic * 8, 8), :, :] += racc

        @pl.when(i == j)
        def _diag_tile():
            rows = lax.broadcasted_iota(jnp.int32, (TI, TJ), 0)
            cols = lax.broadcasted_iota(jnp.int32, (TI, TJ), 1)
            m = (cols <= rows).astype(jnp.float32)
            _accumulate(
                lambda ic, jc: x2[8 * ic:8 * ic + 8, 8 * jc:8 * jc + 8, :]
                * m[8 * ic:8 * ic + 8, 8 * jc:8 * jc + 8][:, :, None])

        @pl.when(i != j)
        def _full_tile():
            _accumulate(lambda ic, jc: x2[8 * ic:8 * ic + 8,
                                          8 * jc:8 * jc + 8, :])

        @pl.when(t == T - 1)
        def _finish():
            s = S_s[...] + jnp.sum(R8_s[...], axis=1)
            u = (jnp.dot(s, W3_ref[...],
                         preferred_element_type=jnp.float32) * inv_count
                 + b3_ref[0])
            sig = lax.dot_general(u, u, (((1,), (1,)), ((), ())),
                                  preferred_element_type=jnp.float32)
            rf = rf_ref[0]
            d_raw = jnp.dot(rf, Wd_ref[...],
                            preferred_element_type=jnp.float32)[:, 0]
            d = _softplus(d_raw + bd_ref[0, 0]) + _EPS
            rr = lax.broadcasted_iota(jnp.int32, (N, N), 0)
            cc = lax.broadcasted_iota(jnp.int32, (N, N), 1)
            sigma_ref[0] = sig + jnp.where(rr == cc, d[:, None], 0.0)

    return body


def kernel(residue_features, attention, W1, b1, W2, b2, W3, b3, Wd, bd):
    B, N, D = residue_features.shape
    HIDDEN = W2.shape[0]
    RANK = W3.shape[1]
    TI = TJ = 128
    nI = N // TI
    nJ = N // TJ

    tril = [(i, j) for i in range(nI) for j in range(nJ) if j <= i]
    T = len(tril)
    imap = jnp.asarray(np.array([p[0] for p in tril], np.int32))
    jmap = jnp.asarray(np.array([p[1] for p in tril], np.int32))

    W1a = W1[:D]
    W1b = W1[D:2 * D]
    w1c = W1[2 * D].reshape(1, HIDDEN)
    b1r = b1.reshape(1, HIDDEN)
    b2r = b2.reshape(1, HIDDEN)
    b3r = b3.reshape(1, RANK)
    bdr = bd.reshape(1, 1)

    body = _make_kernel(N, D, HIDDEN, RANK, TI, TJ, T)

    full = lambda *shape: pl.BlockSpec(
        shape, lambda b, t, im, jm: (0,) * len(shape))

    grid_spec = pltpu.PrefetchScalarGridSpec(
        num_scalar_prefetch=2,
        grid=(B, T),
        in_specs=[
            pl.BlockSpec((1, N, D), lambda b, t, im, jm: (b, 0, 0)),
            pl.BlockSpec((1, TI, TJ),
                         lambda b, t, im, jm: (b, im[t], jm[t])),
            full(D, HIDDEN),
            full(D, HIDDEN),
            full(1, HIDDEN),
            full(1, HIDDEN),
            full(HIDDEN, HIDDEN),
            full(1, HIDDEN),
            full(HIDDEN, RANK),
            full(1, RANK),
            full(D, 1),
            full(1, 1),
        ],
        out_specs=pl.BlockSpec((1, N, N), lambda b, t, im, jm: (b, 0, 0)),
        scratch_shapes=[
            pltpu.VMEM((N, HIDDEN), jnp.float32),
            pltpu.VMEM((N, HIDDEN), jnp.float32),
            pltpu.VMEM((N, HIDDEN), jnp.float32),
            pltpu.VMEM((N, 8, HIDDEN), jnp.float32),
        ],
    )

    out = pl.pallas_call(
        body,
        grid_spec=grid_spec,
        out_shape=jax.ShapeDtypeStruct((B, N, N), jnp.float32),
        compiler_params=pltpu.CompilerParams(
            dimension_semantics=("arbitrary", "arbitrary"),
        ),
    )(imap, jmap,
      residue_features, attention, W1a, W1b, w1c, b1r,
      W2, b2r, W3, b3r, Wd, bdr)
    return out
